# Initial kernel scaffold; baseline (speedup 1.0000x reference)
import jax
import jax.numpy as jnp
from jax.experimental import pallas as pl


def kernel(x, edge_attr, W_feat, b_feat, W_msg, b_msg, edge_index):
    raise NotImplementedError("write your pallas kernel here")


# speedup vs baseline: 1.3740x; 1.3740x over previous
import functools

import numpy as np

import jax
import jax.numpy as jnp
from jax import lax
from jax.experimental import pallas as pl
from jax.experimental.pallas import tpu as pltpu, tpu_sc as plsc

N_NODES = 10000
N_EDGES = 320000
G_FEAT = 128
G_EDGE = 16
EMB = 256

NC = 2
NS = 16
NW = NC * NS
LANES = 16

NODE_BLK = 1000
EDGE_BLK = 4000
PER_W = N_EDGES // NW
CHUNK = 40
N_CHUNKS = PER_W // CHUNK

_PERM = np.empty((EMB,), dtype=np.int32)
for _j in range(EMB // 32):
    for _k in range(16):
        _PERM[32 * _j + 2 * _k] = 32 * _j + _k
        _PERM[32 * _j + 2 * _k + 1] = 32 * _j + 16 + _k


def _node_body(x_ref, wf_ref, bf_ref, w1_ref, w2_ref, vf_ref, p1_ref, p2_ref):
    h = jnp.dot(x_ref[...], wf_ref[...], preferred_element_type=jnp.float32)
    h = jnp.maximum(h + bf_ref[...], 0.0)
    vf_ref[...] = h
    p1_ref[...] = jnp.dot(h, w1_ref[...], preferred_element_type=jnp.float32)
    p2_ref[...] = jnp.dot(h, w2_ref[...], preferred_element_type=jnp.float32)


def _node_embed(x, W_feat, b_feat, W1, W2):
    grid = (N_NODES // NODE_BLK,)
    shp = jax.ShapeDtypeStruct((N_NODES, EMB), jnp.float32)

    return pl.pallas_call(
        _node_body,
        grid=grid,
        in_specs=[
            pl.BlockSpec((NODE_BLK, G_FEAT), lambda i: (i, 0)),
            pl.BlockSpec((G_FEAT, EMB), lambda i: (0, 0)),
            pl.BlockSpec((1, EMB), lambda i: (0, 0)),
            pl.BlockSpec((EMB, EMB), lambda i: (0, 0)),
            pl.BlockSpec((EMB, EMB), lambda i: (0, 0)),
        ],
        out_specs=[
            pl.BlockSpec((NODE_BLK, EMB), lambda i: (i, 0)),
            pl.BlockSpec((NODE_BLK, EMB), lambda i: (i, 0)),
            pl.BlockSpec((NODE_BLK, EMB), lambda i: (i, 0)),
        ],
        out_shape=[shp, shp, shp],
    )(x, W_feat, b_feat, W1, W2)


def _edge_body(e_ref, w3_ref, bm_ref, out_ref):
    e = jnp.maximum(e_ref[...], 0.0)
    out_ref[...] = (
        jnp.dot(e, w3_ref[...], preferred_element_type=jnp.float32) + bm_ref[...]
    )


def _edge_term(edge_attr, W3, b_msg):
    grid = (N_EDGES // EDGE_BLK,)
    return pl.pallas_call(
        _edge_body,
        grid=grid,
        in_specs=[
            pl.BlockSpec((EDGE_BLK, G_EDGE), lambda i: (i, 0)),
            pl.BlockSpec((G_EDGE, EMB), lambda i: (0, 0)),
            pl.BlockSpec((1, EMB), lambda i: (0, 0)),
        ],
        out_specs=pl.BlockSpec((EDGE_BLK, EMB), lambda i: (i, 0)),
        out_shape=jax.ShapeDtypeStruct((N_EDGES, EMB), jnp.float32),
    )(edge_attr, W3, b_msg)


def _combine_body(ewb_hbm, p1_hbm, p2_hbm, src_hbm, dst_hbm, out_hbm,
                  src_v, dst_v, acc_v, g1_v, g2_v,
                  sem_in0, sem_in1, sem_out0, sem_out1):
    wid = lax.axis_index("s") * NC + lax.axis_index("c")
    base = wid * PER_W
    sem_in = (sem_in0, sem_in1)
    sem_out = (sem_out0, sem_out1)
    pltpu.sync_copy(src_hbm.at[pl.ds(base, PER_W)], src_v)
    pltpu.sync_copy(dst_hbm.at[pl.ds(base, PER_W)], dst_v)

    def in_copies(c, b):
        cb = base + c * CHUNK
        return (
            pltpu.make_async_copy(ewb_hbm.at[pl.ds(cb, CHUNK)],
                                  acc_v.at[b], sem_in[b]),
            pltpu.make_async_copy(p1_hbm.at[src_v.at[pl.ds(c * CHUNK, CHUNK)]],
                                  g1_v.at[b], sem_in[b]),
            pltpu.make_async_copy(p2_hbm.at[dst_v.at[pl.ds(c * CHUNK, CHUNK)]],
                                  g2_v.at[b], sem_in[b]),
        )

    def out_copy(c, b):
        cb = base + c * CHUNK
        return pltpu.make_async_copy(acc_v.at[b],
                                     out_hbm.at[pl.ds(cb, CHUNK)], sem_out[b])

    def issue_in(c, b):
        for cp in in_copies(c, b):
            cp.start()

    def wait_in(c, b):
        for cp in in_copies(c, b):
            cp.wait()

    issue_in(0, 0)

    def outer_body(i, carry):
        for b in (0, 1):
            c = 2 * i + b

            @pl.when(c + 1 < N_CHUNKS)
            def _():
                @pl.when(c >= 1)
                def _():
                    out_copy(c - 1, 1 - b).wait()

                issue_in(c + 1, 1 - b)

            wait_in(c, b)

            def row_body(r, carry2):
                hi_mask = jnp.full((16,), -65536, dtype=jnp.int32)
                for j in range(EMB // 32):
                    s16 = pl.ds(16 * j, 16)
                    w1 = g1_v[b, r, s16]
                    w2 = g2_v[b, r, s16]
                    a1 = lax.bitcast_convert_type(w1 << 16, jnp.float32)
                    b1 = lax.bitcast_convert_type(w1 & hi_mask, jnp.float32)
                    a2 = lax.bitcast_convert_type(w2 << 16, jnp.float32)
                    b2 = lax.bitcast_convert_type(w2 & hi_mask, jnp.float32)
                    plsc.addupdate(acc_v.at[b, r, pl.ds(32 * j, LANES)],
                                   a1 + a2)
                    plsc.addupdate(acc_v.at[b, r, pl.ds(32 * j + 16, LANES)],
                                   b1 + b2)
                return carry2

            lax.fori_loop(0, CHUNK, row_body, 0)
            out_copy(c, b).start()
        return carry

    lax.fori_loop(0, N_CHUNKS // 2, outer_body, 0)
    out_copy(N_CHUNKS - 2, 0).wait()
    out_copy(N_CHUNKS - 1, 1).wait()


_combine = functools.partial(
    pl.kernel,
    mesh=plsc.VectorSubcoreMesh(core_axis_name="c", subcore_axis_name="s"),
    out_type=jax.ShapeDtypeStruct((N_EDGES, EMB), jnp.float32),
    scratch_types=[
        pltpu.VMEM((PER_W,), jnp.int32),
        pltpu.VMEM((PER_W,), jnp.int32),
        pltpu.VMEM((2, CHUNK, EMB), jnp.float32),
        pltpu.VMEM((2, CHUNK, EMB // 2), jnp.int32),
        pltpu.VMEM((2, CHUNK, EMB // 2), jnp.int32),
        pltpu.SemaphoreType.DMA,
        pltpu.SemaphoreType.DMA,
        pltpu.SemaphoreType.DMA,
        pltpu.SemaphoreType.DMA,
    ],
)(_combine_body)


def _pack_bf16_pairs(p):
    u = lax.bitcast_convert_type(p, jnp.uint32)
    r = (u + jnp.uint32(0x7FFF) + ((u >> 16) & jnp.uint32(1))) >> 16
    w = r[:, 0::2] | (r[:, 1::2] << 16)
    return lax.bitcast_convert_type(w, jnp.int32)


def kernel(x, edge_attr, W_feat, b_feat, W_msg, b_msg, edge_index):
    perm = jnp.asarray(_PERM)
    W1p = W_msg[:EMB][:, perm]
    W2p = W_msg[EMB:2 * EMB][:, perm]
    W3 = W_msg[2 * EMB:]
    vf, p1, p2 = _node_embed(x, W_feat, b_feat.reshape(1, EMB), W1p, W2p)
    p1i = _pack_bf16_pairs(p1)
    p2i = _pack_bf16_pairs(p2)
    ewb = _edge_term(edge_attr, W3, b_msg.reshape(1, EMB))
    msg_emb = _combine(ewb, p1i, p2i, edge_index[0], edge_index[1])
    return (vf, msg_emb)
